# baseline (speedup 1.0000x reference)
import functools

import jax
import jax.numpy as jnp
from jax import lax
from jax.experimental import pallas as pl
from jax.experimental.pallas import tpu as pltpu
from jax.experimental.pallas import tpu_sc as plsc

N = 10000
D = 128
DW = D // 2
E = 320000
L = 16
NW = 32
EPW = E // NW
CE = 64
NCH = EPW // CE
TAIL = EPW - NCH * CE
RPT = N // 16
RB = 125
NB = RPT // RB
_MAGIC = 0x5F3759DF


def _sc_body(h_hbm, idx_hbm, out_hbm, table_hbm,
             hrows_v, pk_v, idx_v,
             rows0, rows1, out_v,
             sem0, sem1, semi):
    cid = lax.axis_index("c")
    sid = lax.axis_index("s")
    wid = sid * 2 + cid
    base_w = wid * EPW
    rows = (rows0, rows1)
    sems = (sem0, sem1)
    laneid = lax.iota(jnp.int32, L)

    cpi = pltpu.async_copy(idx_hbm.at[pl.ds(base_w * 2, 2 * EPW)], idx_v, semi)

    def block_body(blk, carry):
        r0 = sid * RPT + blk * RB
        pltpu.sync_copy(h_hbm.at[pl.ds(r0, RB)], hrows_v)

        def row_body(r, c):
            xs = [hrows_v[r, pl.ds(k * L, L)] for k in range(D // L)]
            acc = xs[0] * xs[0]
            for k in range(1, D // L):
                acc = acc + xs[k] * xs[k]
            n2v = jnp.maximum(jnp.full((L,), jnp.sum(acc)), 1e-24)
            yi = _MAGIC - (plsc.bitcast(n2v, jnp.int32) >> 1)
            y = plsc.bitcast(yi, jnp.float32)
            xh = 0.5 * n2v
            y = y * (1.5 - xh * y * y)
            y = y * (1.5 - xh * y * y)
            s = jnp.minimum(y, 1.0)
            for k in range(DW // L):
                w = plsc.pack(xs[2 * k] * s, xs[2 * k + 1] * s,
                              format=plsc.PackFormat.INTERLEAVED)
                pk_v[r, pl.ds(k * L, L)] = plsc.bitcast(w, jnp.int32)
            return c

        lax.fori_loop(0, RB, row_body, 0)
        pltpu.sync_copy(pk_v, table_hbm.at[pl.ds(r0, RB)])
        return carry

    lax.fori_loop(0, NB, block_body, 0)
    plsc.subcore_barrier()

    cpi.wait()

    def issue(j, b, nidx):
        off = pl.ds(j * (2 * CE), nidx)
        pltpu.async_copy(table_hbm.at[idx_v.at[off]],
                         rows[b].at[pl.ds(0, nidx)], sems[b])

    def drain(j, b, nidx):
        off = pl.ds(j * (2 * CE), nidx)
        pltpu.make_async_copy(table_hbm.at[idx_v.at[off]],
                              rows[b].at[pl.ds(0, nidx)], sems[b]).wait()

    def compute(j, b, nedges):
        ra = rows[b]

        def group_body(g, c):
            res = jnp.zeros((L,), jnp.float32)
            for i in range(L):
                e = 2 * (g * L + i)
                acc16 = jnp.zeros((2 * L,), jnp.bfloat16)
                for k in range(DW // L):
                    va = plsc.bitcast(ra[e, pl.ds(k * L, L)], jnp.bfloat16)
                    vb = plsc.bitcast(ra[e + 1, pl.ds(k * L, L)], jnp.bfloat16)
                    dv = va - vb
                    acc16 = acc16 + dv * dv
                lo, hi = plsc.unpack(acc16, format=plsc.PackFormat.INTERLEAVED)
                res = jnp.where(laneid == i, jnp.sum(lo + hi), res)
            out_v[pl.ds(j * CE + g * L, L)] = 1.0 / (jnp.exp(res - 2.0) + 1.0)
            return c

        lax.fori_loop(0, nedges // L, group_body, 0)

    issue(0, 0, 2 * CE)

    def pair_body(jj, c):
        for b in (0, 1):
            j = 2 * jj + b
            nxt = 1 - b

            @pl.when(j < NCH)
            def _():
                @pl.when(j + 1 < NCH)
                def _():
                    issue(j + 1, nxt, 2 * CE)

                @pl.when(j + 1 == NCH)
                def _():
                    issue(NCH, nxt, 2 * TAIL)

                drain(j, b, 2 * CE)
                compute(j, b, CE)

        return c

    lax.fori_loop(0, NCH // 2, pair_body, 0)
    drain(NCH, NCH % 2, 2 * TAIL)
    compute(NCH, NCH % 2, TAIL)
    pltpu.sync_copy(out_v, out_hbm.at[pl.ds(base_w, EPW)])


_sc_call = functools.partial(
    pl.kernel,
    mesh=plsc.VectorSubcoreMesh(core_axis_name="c", subcore_axis_name="s"),
    compiler_params=pltpu.CompilerParams(
        needs_layout_passes=False, use_tc_tiling_on_sc=False),
    out_type=(
        jax.ShapeDtypeStruct((E,), jnp.float32),
        jax.ShapeDtypeStruct((N, DW), jnp.int32),
    ),
    scratch_types=[
        pltpu.VMEM((RB, D), jnp.float32),
        pltpu.VMEM((RB, DW), jnp.int32),
        pltpu.VMEM((2 * EPW,), jnp.int32),
        pltpu.VMEM((2 * CE, DW), jnp.int32),
        pltpu.VMEM((2 * CE, DW), jnp.int32),
        pltpu.VMEM((EPW,), jnp.float32),
        pltpu.SemaphoreType.DMA,
        pltpu.SemaphoreType.DMA,
        pltpu.SemaphoreType.DMA,
    ],
)(_sc_body)


def kernel(h, idx):
    probs, _ = _sc_call(h, idx.reshape(-1))
    return probs


# speedup vs baseline: 2.0326x; 2.0326x over previous
import functools

import jax
import jax.numpy as jnp
from jax import lax
from jax.experimental import pallas as pl
from jax.experimental.pallas import tpu as pltpu
from jax.experimental.pallas import tpu_sc as plsc

N = 10000
D = 128
DW = D // 2
E = 320000
L = 16
NW = 32
EPW = E // NW
CH = 80
NCH = EPW // CH
RPT = N // 16
RB = 125
NB = RPT // RB
_MAGIC = 0x5F3759DF


def _sc_body(h_hbm, idx0_hbm, idx1_hbm, out_hbm, table_hbm,
             hrows_v, pk_v, idx0_v, idx1_v,
             rows_a0, rows_b0, rows_a1, rows_b1, out_v,
             sem0, sem1, semn):
    cid = lax.axis_index("c")
    sid = lax.axis_index("s")
    wid = sid * 2 + cid
    base_w = wid * EPW
    rows_a = (rows_a0, rows_a1)
    rows_b = (rows_b0, rows_b1)
    sems = (sem0, sem1)
    laneid = lax.iota(jnp.int32, L)

    cp0 = pltpu.async_copy(idx0_hbm.at[pl.ds(base_w, EPW)], idx0_v, sem0)
    cp1 = pltpu.async_copy(idx1_hbm.at[pl.ds(base_w, EPW)], idx1_v, sem1)

    def block_body(blk, carry):
        r0 = sid * RPT + blk * RB
        pltpu.sync_copy(h_hbm.at[pl.ds(r0, RB)], hrows_v)

        def row_body(r, c):
            xs = [hrows_v[r, pl.ds(k * L, L)] for k in range(D // L)]
            acc = xs[0] * xs[0]
            for k in range(1, D // L):
                acc = acc + xs[k] * xs[k]
            n2v = jnp.maximum(jnp.full((L,), jnp.sum(acc)), 1e-24)
            yi = _MAGIC - (plsc.bitcast(n2v, jnp.int32) >> 1)
            y = plsc.bitcast(yi, jnp.float32)
            xh = 0.5 * n2v
            y = y * (1.5 - xh * y * y)
            y = y * (1.5 - xh * y * y)
            s = jnp.minimum(y, 1.0)
            for k in range(DW // L):
                w = plsc.pack(xs[2 * k] * s, xs[2 * k + 1] * s,
                              format=plsc.PackFormat.INTERLEAVED)
                pk_v[r, pl.ds(k * L, L)] = plsc.bitcast(w, jnp.int32)
            return c

        lax.fori_loop(0, RB, row_body, 0)
        pltpu.sync_copy(pk_v, table_hbm.at[pl.ds(r0, RB)])
        return carry

    lax.fori_loop(0, NB, block_body, 0)
    plsc.subcore_barrier()

    cp0.wait()
    cp1.wait()

    CH2 = CH // 2

    def issue(j, b):
        o1 = pl.ds(j * CH, CH2)
        o2 = pl.ds(j * CH + CH2, CH2)
        lo = pl.ds(0, CH2)
        hi = pl.ds(CH2, CH2)
        pltpu.async_copy(table_hbm.at[idx0_v.at[o1]], rows_a[b].at[lo], sems[b])
        pltpu.async_copy(table_hbm.at[idx0_v.at[o2]], rows_a[b].at[hi], sems[b])
        pltpu.async_copy(table_hbm.at[idx1_v.at[o1]], rows_b[b].at[lo], sems[b])
        pltpu.async_copy(table_hbm.at[idx1_v.at[o2]], rows_b[b].at[hi], sems[b])

    def drain(j, b):
        o1 = pl.ds(j * CH, CH2)
        o2 = pl.ds(j * CH + CH2, CH2)
        lo = pl.ds(0, CH2)
        hi = pl.ds(CH2, CH2)
        pltpu.make_async_copy(table_hbm.at[idx0_v.at[o1]], rows_a[b].at[lo], sems[b]).wait()
        pltpu.make_async_copy(table_hbm.at[idx0_v.at[o2]], rows_a[b].at[hi], sems[b]).wait()
        pltpu.make_async_copy(table_hbm.at[idx1_v.at[o1]], rows_b[b].at[lo], sems[b]).wait()
        pltpu.make_async_copy(table_hbm.at[idx1_v.at[o2]], rows_b[b].at[hi], sems[b]).wait()

    def compute(j, b):
        ra, rb = rows_a[b], rows_b[b]

        def group_body(g, c):
            res = jnp.zeros((L,), jnp.float32)
            for i in range(L):
                e = g * L + i
                acc16 = jnp.zeros((2 * L,), jnp.bfloat16)
                for k in range(DW // L):
                    va = plsc.bitcast(ra[e, pl.ds(k * L, L)], jnp.bfloat16)
                    vb = plsc.bitcast(rb[e, pl.ds(k * L, L)], jnp.bfloat16)
                    dv = va - vb
                    acc16 = acc16 + dv * dv
                lo, hi = plsc.unpack(acc16, format=plsc.PackFormat.INTERLEAVED)
                res = jnp.where(laneid == i, jnp.sum(lo + hi), res)
            out_v[pl.ds(j * CH + g * L, L)] = 1.0 / (jnp.exp(res - 2.0) + 1.0)
            return c

        lax.fori_loop(0, CH // L, group_body, 0)

    issue(0, 0)

    def pair_body(jj, c):
        for b in (0, 1):
            j = 2 * jj + b
            nb = 1 - b

            @pl.when(j < NCH)
            def _():
                @pl.when(j + 1 < NCH)
                def _():
                    issue(j + 1, nb)

                drain(j, b)
                compute(j, b)

        return c

    lax.fori_loop(0, (NCH + 1) // 2, pair_body, 0)
    pltpu.sync_copy(out_v, out_hbm.at[pl.ds(base_w, EPW)])


_sc_call = functools.partial(
    pl.kernel,
    mesh=plsc.VectorSubcoreMesh(core_axis_name="c", subcore_axis_name="s"),
    compiler_params=pltpu.CompilerParams(
        needs_layout_passes=False, use_tc_tiling_on_sc=False),
    out_type=(
        jax.ShapeDtypeStruct((E,), jnp.float32),
        jax.ShapeDtypeStruct((N, DW), jnp.int32),
    ),
    scratch_types=[
        pltpu.VMEM((RB, D), jnp.float32),
        pltpu.VMEM((RB, DW), jnp.int32),
        pltpu.VMEM((EPW,), jnp.int32),
        pltpu.VMEM((EPW,), jnp.int32),
        pltpu.VMEM((CH, DW), jnp.int32),
        pltpu.VMEM((CH, DW), jnp.int32),
        pltpu.VMEM((CH, DW), jnp.int32),
        pltpu.VMEM((CH, DW), jnp.int32),
        pltpu.VMEM((EPW,), jnp.float32),
        pltpu.SemaphoreType.DMA,
        pltpu.SemaphoreType.DMA,
        pltpu.SemaphoreType.DMA,
    ],
)(_sc_body)


def kernel(h, idx):
    probs, _ = _sc_call(h, idx[:, 0], idx[:, 1])
    return probs


# speedup vs baseline: 2.0332x; 1.0003x over previous
import functools

import jax
import jax.numpy as jnp
from jax import lax
from jax.experimental import pallas as pl
from jax.experimental.pallas import tpu as pltpu
from jax.experimental.pallas import tpu_sc as plsc

N = 10000
D = 128
DW = D // 2
E = 320000
L = 16
NW = 32
EPW = E // NW
CH = 80
NCH = EPW // CH
RPT = N // 16
RB = 125
NB = RPT // RB
_MAGIC = 0x5F3759DF


def _sc_body(h_hbm, idx0_hbm, idx1_hbm, out_hbm, table_hbm,
             hrows_v, pk_v, idx0_v, idx1_v,
             rows_a0, rows_b0, rows_a1, rows_b1, out_v,
             sem0, sem1, semn):
    cid = lax.axis_index("c")
    sid = lax.axis_index("s")
    wid = sid * 2 + cid
    base_w = wid * EPW
    rows_a = (rows_a0, rows_a1)
    rows_b = (rows_b0, rows_b1)
    sems = (sem0, sem1)
    laneid = lax.iota(jnp.int32, L)

    cp0 = pltpu.async_copy(idx0_hbm.at[pl.ds(base_w, EPW)], idx0_v, sem0)
    cp1 = pltpu.async_copy(idx1_hbm.at[pl.ds(base_w, EPW)], idx1_v, sem1)

    def block_body(blk, carry):
        r0 = sid * RPT + blk * RB
        pltpu.sync_copy(h_hbm.at[pl.ds(r0, RB)], hrows_v)

        def row_body(rq, c):
            for u in range(5):
                r = rq * 5 + u
                xs = [hrows_v[r, pl.ds(k * L, L)] for k in range(D // L)]
                acc = xs[0] * xs[0]
                for k in range(1, D // L):
                    acc = acc + xs[k] * xs[k]
                n2v = jnp.maximum(jnp.full((L,), jnp.sum(acc)), 1e-24)
                yi = _MAGIC - (plsc.bitcast(n2v, jnp.int32) >> 1)
                y = plsc.bitcast(yi, jnp.float32)
                xh = 0.5 * n2v
                y = y * (1.5 - xh * y * y)
                y = y * (1.5 - xh * y * y)
                s = jnp.minimum(y, 1.0)
                for k in range(DW // L):
                    w = plsc.pack(xs[2 * k] * s, xs[2 * k + 1] * s,
                                  format=plsc.PackFormat.INTERLEAVED)
                    pk_v[r, pl.ds(k * L, L)] = plsc.bitcast(w, jnp.int32)
            return c

        lax.fori_loop(0, RB // 5, row_body, 0)
        pltpu.sync_copy(pk_v, table_hbm.at[pl.ds(r0, RB)])
        return carry

    lax.fori_loop(0, NB, block_body, 0)
    plsc.subcore_barrier()

    cp0.wait()
    cp1.wait()

    CH2 = CH // 2

    def issue(j, b):
        o1 = pl.ds(j * CH, CH2)
        o2 = pl.ds(j * CH + CH2, CH2)
        lo = pl.ds(0, CH2)
        hi = pl.ds(CH2, CH2)
        pltpu.async_copy(table_hbm.at[idx0_v.at[o1]], rows_a[b].at[lo], sems[b])
        pltpu.async_copy(table_hbm.at[idx0_v.at[o2]], rows_a[b].at[hi], sems[b])
        pltpu.async_copy(table_hbm.at[idx1_v.at[o1]], rows_b[b].at[lo], sems[b])
        pltpu.async_copy(table_hbm.at[idx1_v.at[o2]], rows_b[b].at[hi], sems[b])

    def drain(j, b):
        o1 = pl.ds(j * CH, CH2)
        o2 = pl.ds(j * CH + CH2, CH2)
        lo = pl.ds(0, CH2)
        hi = pl.ds(CH2, CH2)
        pltpu.make_async_copy(table_hbm.at[idx0_v.at[o1]], rows_a[b].at[lo], sems[b]).wait()
        pltpu.make_async_copy(table_hbm.at[idx0_v.at[o2]], rows_a[b].at[hi], sems[b]).wait()
        pltpu.make_async_copy(table_hbm.at[idx1_v.at[o1]], rows_b[b].at[lo], sems[b]).wait()
        pltpu.make_async_copy(table_hbm.at[idx1_v.at[o2]], rows_b[b].at[hi], sems[b]).wait()

    def compute(j, b):
        ra, rb = rows_a[b], rows_b[b]

        def group_body(g, c):
            res = jnp.zeros((L,), jnp.float32)
            for i in range(L):
                e = g * L + i
                acc16 = jnp.zeros((2 * L,), jnp.bfloat16)
                for k in range(DW // L):
                    va = plsc.bitcast(ra[e, pl.ds(k * L, L)], jnp.bfloat16)
                    vb = plsc.bitcast(rb[e, pl.ds(k * L, L)], jnp.bfloat16)
                    dv = va - vb
                    acc16 = acc16 + dv * dv
                lo, hi = plsc.unpack(acc16, format=plsc.PackFormat.INTERLEAVED)
                res = jnp.where(laneid == i, jnp.sum(lo + hi), res)
            out_v[pl.ds(j * CH + g * L, L)] = 1.0 / (jnp.exp(res - 2.0) + 1.0)
            return c

        lax.fori_loop(0, CH // L, group_body, 0)

    issue(0, 0)

    def pair_body(jj, c):
        for b in (0, 1):
            j = 2 * jj + b
            nb = 1 - b

            @pl.when(j < NCH)
            def _():
                @pl.when(j + 1 < NCH)
                def _():
                    issue(j + 1, nb)

                drain(j, b)
                compute(j, b)

        return c

    lax.fori_loop(0, (NCH + 1) // 2, pair_body, 0)
    pltpu.sync_copy(out_v, out_hbm.at[pl.ds(base_w, EPW)])


_sc_call = functools.partial(
    pl.kernel,
    mesh=plsc.VectorSubcoreMesh(core_axis_name="c", subcore_axis_name="s"),
    compiler_params=pltpu.CompilerParams(
        needs_layout_passes=False, use_tc_tiling_on_sc=False),
    out_type=(
        jax.ShapeDtypeStruct((E,), jnp.float32),
        jax.ShapeDtypeStruct((N, DW), jnp.int32),
    ),
    scratch_types=[
        pltpu.VMEM((RB, D), jnp.float32),
        pltpu.VMEM((RB, DW), jnp.int32),
        pltpu.VMEM((EPW,), jnp.int32),
        pltpu.VMEM((EPW,), jnp.int32),
        pltpu.VMEM((CH, DW), jnp.int32),
        pltpu.VMEM((CH, DW), jnp.int32),
        pltpu.VMEM((CH, DW), jnp.int32),
        pltpu.VMEM((CH, DW), jnp.int32),
        pltpu.VMEM((EPW,), jnp.float32),
        pltpu.SemaphoreType.DMA,
        pltpu.SemaphoreType.DMA,
        pltpu.SemaphoreType.DMA,
    ],
)(_sc_body)


def kernel(h, idx):
    probs, _ = _sc_call(h, idx[:, 0], idx[:, 1])
    return probs


# speedup vs baseline: 2.1009x; 1.0333x over previous
import functools

import jax
import jax.numpy as jnp
from jax import lax
from jax.experimental import pallas as pl
from jax.experimental.pallas import tpu as pltpu
from jax.experimental.pallas import tpu_sc as plsc

N = 10000
D = 128
DW = D // 2
E = 320000
L = 16
NW = 32
EPW = E // NW
CH = 80
NCH = EPW // CH
RPT = N // 16
RB = 125
NB = RPT // RB
_MAGIC = 0x5F3759DF


def _sc_body(h_hbm, idx0_hbm, idx1_hbm, out_hbm, table_hbm,
             hrows_v0, hrows_v1, pk_v0, pk_v1, idx0_v, idx1_v,
             rows_a0, rows_b0, rows_a1, rows_b1, out_v,
             sem0, sem1, lsem0, lsem1, ssem0, ssem1):
    cid = lax.axis_index("c")
    sid = lax.axis_index("s")
    wid = sid * 2 + cid
    base_w = wid * EPW
    rows_a = (rows_a0, rows_a1)
    rows_b = (rows_b0, rows_b1)
    sems = (sem0, sem1)
    laneid = lax.iota(jnp.int32, L)

    cp0 = pltpu.async_copy(idx0_hbm.at[pl.ds(base_w, EPW)], idx0_v, sem0)
    cp1 = pltpu.async_copy(idx1_hbm.at[pl.ds(base_w, EPW)], idx1_v, sem1)

    hrows = (hrows_v0, hrows_v1)
    pk = (pk_v0, pk_v1)
    lsem = (lsem0, lsem1)
    ssem = (ssem0, ssem1)

    def blk_rows(blk):
        return pl.ds(sid * RPT + blk * RB, RB)

    def do_block(hr, pkv):
        def row_body(rq, c):
            for u in range(5):
                r = rq * 5 + u
                xs = [hr[r, pl.ds(k * L, L)] for k in range(D // L)]
                acc = xs[0] * xs[0]
                for k in range(1, D // L):
                    acc = acc + xs[k] * xs[k]
                n2v = jnp.maximum(jnp.full((L,), jnp.sum(acc)), 1e-24)
                yi = _MAGIC - (plsc.bitcast(n2v, jnp.int32) >> 1)
                y = plsc.bitcast(yi, jnp.float32)
                xh = 0.5 * n2v
                y = y * (1.5 - xh * y * y)
                y = y * (1.5 - xh * y * y)
                s = jnp.minimum(y, 1.0)
                for k in range(DW // L):
                    w = plsc.pack(xs[2 * k] * s, xs[2 * k + 1] * s,
                                  format=plsc.PackFormat.INTERLEAVED)
                    pkv[r, pl.ds(k * L, L)] = plsc.bitcast(w, jnp.int32)
            return c

        lax.fori_loop(0, RB // 5, row_body, 0)

    pltpu.async_copy(h_hbm.at[blk_rows(0)], hrows[0], lsem[0])
    for blk in range(NB):
        b = blk % 2
        if blk + 1 < NB:
            pltpu.async_copy(h_hbm.at[blk_rows(blk + 1)],
                             hrows[1 - b], lsem[1 - b])
        pltpu.make_async_copy(h_hbm.at[blk_rows(blk)],
                              hrows[b], lsem[b]).wait()
        if blk >= 2:
            pltpu.make_async_copy(pk[b], table_hbm.at[blk_rows(blk - 2)],
                                  ssem[b]).wait()
        do_block(hrows[b], pk[b])
        pltpu.async_copy(pk[b], table_hbm.at[blk_rows(blk)], ssem[b])

    pltpu.make_async_copy(pk[(NB - 2) % 2], table_hbm.at[blk_rows(NB - 2)],
                          ssem[(NB - 2) % 2]).wait()
    pltpu.make_async_copy(pk[(NB - 1) % 2], table_hbm.at[blk_rows(NB - 1)],
                          ssem[(NB - 1) % 2]).wait()
    plsc.subcore_barrier()

    cp0.wait()
    cp1.wait()

    CH2 = CH // 2

    def issue(j, b):
        o1 = pl.ds(j * CH, CH2)
        o2 = pl.ds(j * CH + CH2, CH2)
        lo = pl.ds(0, CH2)
        hi = pl.ds(CH2, CH2)
        pltpu.async_copy(table_hbm.at[idx0_v.at[o1]], rows_a[b].at[lo], sems[b])
        pltpu.async_copy(table_hbm.at[idx0_v.at[o2]], rows_a[b].at[hi], sems[b])
        pltpu.async_copy(table_hbm.at[idx1_v.at[o1]], rows_b[b].at[lo], sems[b])
        pltpu.async_copy(table_hbm.at[idx1_v.at[o2]], rows_b[b].at[hi], sems[b])

    def drain(j, b):
        o1 = pl.ds(j * CH, CH2)
        o2 = pl.ds(j * CH + CH2, CH2)
        lo = pl.ds(0, CH2)
        hi = pl.ds(CH2, CH2)
        pltpu.make_async_copy(table_hbm.at[idx0_v.at[o1]], rows_a[b].at[lo], sems[b]).wait()
        pltpu.make_async_copy(table_hbm.at[idx0_v.at[o2]], rows_a[b].at[hi], sems[b]).wait()
        pltpu.make_async_copy(table_hbm.at[idx1_v.at[o1]], rows_b[b].at[lo], sems[b]).wait()
        pltpu.make_async_copy(table_hbm.at[idx1_v.at[o2]], rows_b[b].at[hi], sems[b]).wait()

    def compute(j, b):
        ra, rb = rows_a[b], rows_b[b]

        def group_body(g, c):
            res = jnp.zeros((L,), jnp.float32)
            for i in range(L):
                e = g * L + i
                acc16 = jnp.zeros((2 * L,), jnp.bfloat16)
                for k in range(DW // L):
                    va = plsc.bitcast(ra[e, pl.ds(k * L, L)], jnp.bfloat16)
                    vb = plsc.bitcast(rb[e, pl.ds(k * L, L)], jnp.bfloat16)
                    dv = va - vb
                    acc16 = acc16 + dv * dv
                lo, hi = plsc.unpack(acc16, format=plsc.PackFormat.INTERLEAVED)
                res = jnp.where(laneid == i, jnp.sum(lo + hi), res)
            out_v[pl.ds(j * CH + g * L, L)] = 1.0 / (jnp.exp(res - 2.0) + 1.0)
            return c

        lax.fori_loop(0, CH // L, group_body, 0)

    issue(0, 0)

    def pair_body(jj, c):
        for b in (0, 1):
            j = 2 * jj + b
            nb = 1 - b

            @pl.when(j < NCH)
            def _():
                @pl.when(j + 1 < NCH)
                def _():
                    issue(j + 1, nb)

                drain(j, b)
                compute(j, b)

        return c

    lax.fori_loop(0, (NCH + 1) // 2, pair_body, 0)
    pltpu.sync_copy(out_v, out_hbm.at[pl.ds(base_w, EPW)])


_sc_call = functools.partial(
    pl.kernel,
    mesh=plsc.VectorSubcoreMesh(core_axis_name="c", subcore_axis_name="s"),
    compiler_params=pltpu.CompilerParams(
        needs_layout_passes=False, use_tc_tiling_on_sc=False),
    out_type=(
        jax.ShapeDtypeStruct((E,), jnp.float32),
        jax.ShapeDtypeStruct((N, DW), jnp.int32),
    ),
    scratch_types=[
        pltpu.VMEM((RB, D), jnp.float32),
        pltpu.VMEM((RB, D), jnp.float32),
        pltpu.VMEM((RB, DW), jnp.int32),
        pltpu.VMEM((RB, DW), jnp.int32),
        pltpu.VMEM((EPW,), jnp.int32),
        pltpu.VMEM((EPW,), jnp.int32),
        pltpu.VMEM((CH, DW), jnp.int32),
        pltpu.VMEM((CH, DW), jnp.int32),
        pltpu.VMEM((CH, DW), jnp.int32),
        pltpu.VMEM((CH, DW), jnp.int32),
        pltpu.VMEM((EPW,), jnp.float32),
        pltpu.SemaphoreType.DMA,
        pltpu.SemaphoreType.DMA,
        pltpu.SemaphoreType.DMA,
        pltpu.SemaphoreType.DMA,
        pltpu.SemaphoreType.DMA,
        pltpu.SemaphoreType.DMA,
    ],
)(_sc_body)


def kernel(h, idx):
    probs, _ = _sc_call(h, idx[:, 0], idx[:, 1])
    return probs


# speedup vs baseline: 2.1231x; 1.0106x over previous
import functools

import jax
import jax.numpy as jnp
from jax import lax
from jax.experimental import pallas as pl
from jax.experimental.pallas import tpu as pltpu
from jax.experimental.pallas import tpu_sc as plsc

N = 10000
D = 128
DW = D // 2
E = 320000
L = 16
NW = 32
EPW = E // NW
CH = 80
NCH = EPW // CH
RPT = N // 16
RB = 125
NB = RPT // RB
_MAGIC = 0x5F3759DF


def _sc_body(h_hbm, idx0_hbm, idx1_hbm, out_hbm, table_hbm,
             hrows_v0, hrows_v1, pk_v0, pk_v1, idx0_v, idx1_v,
             rows_a0, rows_b0, rows_a1, rows_b1, out_v,
             sem0, sem1, lsem0, lsem1, ssem0, ssem1):
    cid = lax.axis_index("c")
    sid = lax.axis_index("s")
    wid = sid * 2 + cid
    base_w = wid * EPW
    rows_a = (rows_a0, rows_a1)
    rows_b = (rows_b0, rows_b1)
    sems = (sem0, sem1)
    laneid = lax.iota(jnp.int32, L)

    cp0 = pltpu.async_copy(idx0_hbm.at[pl.ds(base_w, EPW)], idx0_v, sem0)
    cp1 = pltpu.async_copy(idx1_hbm.at[pl.ds(base_w, EPW)], idx1_v, sem1)

    hrows = (hrows_v0, hrows_v1)
    pk = (pk_v0, pk_v1)
    lsem = (lsem0, lsem1)
    ssem = (ssem0, ssem1)

    def blk_rows(blk):
        return pl.ds(sid * RPT + blk * RB, RB)

    def do_block(hr, pkv):
        def row_body(rq, c):
            for u in range(5):
                r = rq * 5 + u
                xs = [hr[r, pl.ds(k * L, L)] for k in range(D // L)]
                acc = xs[0] * xs[0]
                for k in range(1, D // L):
                    acc = acc + xs[k] * xs[k]
                n2v = jnp.maximum(jnp.full((L,), jnp.sum(acc)), 1e-24)
                yi = _MAGIC - (plsc.bitcast(n2v, jnp.int32) >> 1)
                y = plsc.bitcast(yi, jnp.float32)
                xh = 0.5 * n2v
                y = y * (1.5 - xh * y * y)
                s = jnp.minimum(y, 1.0)
                for k in range(DW // L):
                    w = plsc.pack(xs[2 * k] * s, xs[2 * k + 1] * s,
                                  format=plsc.PackFormat.INTERLEAVED)
                    pkv[r, pl.ds(k * L, L)] = plsc.bitcast(w, jnp.int32)
            return c

        lax.fori_loop(0, RB // 5, row_body, 0)

    pltpu.async_copy(h_hbm.at[blk_rows(0)], hrows[0], lsem[0])
    for blk in range(NB):
        b = blk % 2
        if blk + 1 < NB:
            pltpu.async_copy(h_hbm.at[blk_rows(blk + 1)],
                             hrows[1 - b], lsem[1 - b])
        pltpu.make_async_copy(h_hbm.at[blk_rows(blk)],
                              hrows[b], lsem[b]).wait()
        if blk >= 2:
            pltpu.make_async_copy(pk[b], table_hbm.at[blk_rows(blk - 2)],
                                  ssem[b]).wait()
        do_block(hrows[b], pk[b])
        pltpu.async_copy(pk[b], table_hbm.at[blk_rows(blk)], ssem[b])

    pltpu.make_async_copy(pk[(NB - 2) % 2], table_hbm.at[blk_rows(NB - 2)],
                          ssem[(NB - 2) % 2]).wait()
    pltpu.make_async_copy(pk[(NB - 1) % 2], table_hbm.at[blk_rows(NB - 1)],
                          ssem[(NB - 1) % 2]).wait()
    plsc.subcore_barrier()

    cp0.wait()
    cp1.wait()

    CH2 = CH // 2

    def issue(j, b):
        o1 = pl.ds(j * CH, CH2)
        o2 = pl.ds(j * CH + CH2, CH2)
        lo = pl.ds(0, CH2)
        hi = pl.ds(CH2, CH2)
        pltpu.async_copy(table_hbm.at[idx0_v.at[o1]], rows_a[b].at[lo], sems[b])
        pltpu.async_copy(table_hbm.at[idx0_v.at[o2]], rows_a[b].at[hi], sems[b])
        pltpu.async_copy(table_hbm.at[idx1_v.at[o1]], rows_b[b].at[lo], sems[b])
        pltpu.async_copy(table_hbm.at[idx1_v.at[o2]], rows_b[b].at[hi], sems[b])

    def drain(j, b):
        o1 = pl.ds(j * CH, CH2)
        o2 = pl.ds(j * CH + CH2, CH2)
        lo = pl.ds(0, CH2)
        hi = pl.ds(CH2, CH2)
        pltpu.make_async_copy(table_hbm.at[idx0_v.at[o1]], rows_a[b].at[lo], sems[b]).wait()
        pltpu.make_async_copy(table_hbm.at[idx0_v.at[o2]], rows_a[b].at[hi], sems[b]).wait()
        pltpu.make_async_copy(table_hbm.at[idx1_v.at[o1]], rows_b[b].at[lo], sems[b]).wait()
        pltpu.make_async_copy(table_hbm.at[idx1_v.at[o2]], rows_b[b].at[hi], sems[b]).wait()

    def compute(j, b):
        ra, rb = rows_a[b], rows_b[b]

        def group_body(g, c):
            res = jnp.zeros((L,), jnp.float32)
            for i in range(L):
                e = g * L + i
                acc16 = jnp.zeros((2 * L,), jnp.bfloat16)
                for k in range(DW // L):
                    va = plsc.bitcast(ra[e, pl.ds(k * L, L)], jnp.bfloat16)
                    vb = plsc.bitcast(rb[e, pl.ds(k * L, L)], jnp.bfloat16)
                    dv = va - vb
                    acc16 = acc16 + dv * dv
                lo, hi = plsc.unpack(acc16, format=plsc.PackFormat.INTERLEAVED)
                res = jnp.where(laneid == i, jnp.sum(lo + hi), res)
            out_v[pl.ds(j * CH + g * L, L)] = 1.0 / (jnp.exp(res - 2.0) + 1.0)
            return c

        lax.fori_loop(0, CH // L, group_body, 0)

    issue(0, 0)

    def pair_body(jj, c):
        for b in (0, 1):
            j = 2 * jj + b
            nb = 1 - b

            @pl.when(j < NCH)
            def _():
                @pl.when(j + 1 < NCH)
                def _():
                    issue(j + 1, nb)

                drain(j, b)
                compute(j, b)

        return c

    lax.fori_loop(0, (NCH + 1) // 2, pair_body, 0)
    pltpu.sync_copy(out_v, out_hbm.at[pl.ds(base_w, EPW)])


_sc_call = functools.partial(
    pl.kernel,
    mesh=plsc.VectorSubcoreMesh(core_axis_name="c", subcore_axis_name="s"),
    compiler_params=pltpu.CompilerParams(
        needs_layout_passes=False, use_tc_tiling_on_sc=False),
    out_type=(
        jax.ShapeDtypeStruct((E,), jnp.float32),
        jax.ShapeDtypeStruct((N, DW), jnp.int32),
    ),
    scratch_types=[
        pltpu.VMEM((RB, D), jnp.float32),
        pltpu.VMEM((RB, D), jnp.float32),
        pltpu.VMEM((RB, DW), jnp.int32),
        pltpu.VMEM((RB, DW), jnp.int32),
        pltpu.VMEM((EPW,), jnp.int32),
        pltpu.VMEM((EPW,), jnp.int32),
        pltpu.VMEM((CH, DW), jnp.int32),
        pltpu.VMEM((CH, DW), jnp.int32),
        pltpu.VMEM((CH, DW), jnp.int32),
        pltpu.VMEM((CH, DW), jnp.int32),
        pltpu.VMEM((EPW,), jnp.float32),
        pltpu.SemaphoreType.DMA,
        pltpu.SemaphoreType.DMA,
        pltpu.SemaphoreType.DMA,
        pltpu.SemaphoreType.DMA,
        pltpu.SemaphoreType.DMA,
        pltpu.SemaphoreType.DMA,
    ],
)(_sc_body)


def kernel(h, idx):
    probs, _ = _sc_call(h, idx[:, 0], idx[:, 1])
    return probs
